# baseline (speedup 1.0000x reference)
import jax
import jax.numpy as jnp
from jax import lax
from jax.experimental import pallas as pl
from jax.experimental.pallas import tpu as pltpu
from jax.experimental.pallas import tpu_sc as plsc

_BATCH = 16384
_FEAT = 64
_NC = 2
_NS = 16
_NW = _NC * _NS
_BPW = _BATCH // _NW
_ICHUNK = 128
_NCHUNK = _BPW // _ICHUNK
_SCALE = 0.5 / _BATCH


def _sc_body(label_hbm, feat_hbm, centers_hbm, part_hbm,
             idx_v, rows_v, feat_v, part_v, sem):
    wid = lax.axis_index("s") * _NC + lax.axis_index("c")
    base = wid * _BPW

    pltpu.sync_copy(label_hbm.at[pl.ds(wid * _NCHUNK, _NCHUNK)], idx_v)

    copies = []
    for j in range(_NCHUNK):
        copies.append(
            pltpu.async_copy(
                centers_hbm.at[idx_v.at[j]],
                rows_v.at[pl.ds(j * _ICHUNK, _ICHUNK)],
                sem,
            )
        )
    pltpu.sync_copy(feat_hbm.at[pl.ds(base, _BPW)], feat_v)
    for cp in copies:
        cp.wait()

    def body(i, accs):
        a0, a1, a2, a3 = accs
        d0 = feat_v[i, pl.ds(0, 16)] - rows_v[i, pl.ds(0, 16)]
        d1 = feat_v[i, pl.ds(16, 16)] - rows_v[i, pl.ds(16, 16)]
        d2 = feat_v[i, pl.ds(32, 16)] - rows_v[i, pl.ds(32, 16)]
        d3 = feat_v[i, pl.ds(48, 16)] - rows_v[i, pl.ds(48, 16)]
        return (a0 + d0 * d0, a1 + d1 * d1, a2 + d2 * d2, a3 + d3 * d3)

    zeros = jnp.zeros((16,), jnp.float32)
    a0, a1, a2, a3 = lax.fori_loop(0, _BPW, body, (zeros, zeros, zeros, zeros))
    part_v[...] = ((a0 + a1) + (a2 + a3)) * _SCALE
    pltpu.sync_copy(part_v, part_hbm.at[wid])


def _finish_body(part_ref, out_ref):
    out_ref[0, 0] = jnp.sum(part_ref[...])


@jax.jit
def kernel(label, feat, centers):
    label2d = label.reshape(_NW * _NCHUNK, _ICHUNK)

    sc = pl.kernel(
        _sc_body,
        out_type=jax.ShapeDtypeStruct((_NW, 16), jnp.float32),
        mesh=plsc.VectorSubcoreMesh(core_axis_name="c", subcore_axis_name="s"),
        compiler_params=pltpu.CompilerParams(use_tc_tiling_on_sc=False),
        scratch_types=[
            pltpu.VMEM((_NCHUNK, _ICHUNK), jnp.int32),
            pltpu.VMEM((_BPW, _FEAT), jnp.float32),
            pltpu.VMEM((_BPW, _FEAT), jnp.float32),
            pltpu.VMEM((16,), jnp.float32),
            pltpu.SemaphoreType.DMA,
        ],
    )
    partials = sc(label2d, feat, centers)

    loss11 = pl.pallas_call(
        _finish_body,
        out_shape=jax.ShapeDtypeStruct((1, 1), jnp.float32),
        out_specs=pl.BlockSpec(memory_space=pltpu.SMEM),
    )(partials)
    return loss11[0, 0]


# speedup vs baseline: 2.4579x; 2.4579x over previous
import jax
import jax.numpy as jnp
from jax import lax
from jax.experimental import pallas as pl
from jax.experimental.pallas import tpu as pltpu
from jax.experimental.pallas import tpu_sc as plsc

_BATCH = 16384
_FEAT = 64
_NCLASS = 1000000
_NC = 2
_NS = 16
_NW = _NC * _NS
_CHUNK = 384
_NCHUNKS = 999936 // _CHUNK
_TAIL0 = 999936
_SCALE = 0.5 / _BATCH
_POSBITS = 14


def _iota16():
    return lax.iota(jnp.int32, 16)


def _sc_body(label_hbm, feat_hbm, centersT_hbm, tail_hbm, part_hbm,
             labels_v, hits_v, clist_v, buf0, buf1, ring_v, tail_v, part_v,
             sem_misc, sem_feat, sem0, sem1):
    w = lax.axis_index("s") * _NC + lax.axis_index("c")
    lo = w * _NCHUNKS // _NW
    hi = (w + 1) * _NCHUNKS // _NW
    base = lo * _CHUNK
    limit = jnp.where(w == _NW - 1, _NCLASS, hi * _CHUNK)

    cp_lab = pltpu.async_copy(label_hbm, labels_v, sem_misc)
    cp_tail = pltpu.async_copy(tail_hbm, tail_v, sem_misc)
    pltpu.async_copy(
        centersT_hbm.at[:, pl.ds(pl.multiple_of(lo * _CHUNK, 128), _CHUNK)],
        buf0, sem0)
    cp_lab.wait()
    cp_tail.wait()

    iota = _iota16()

    def scan_body(j, cnt):
        labv = labels_v[pl.ds(j * 16, 16)]
        m = (labv >= base) & (labv < limit)
        packed = lax.shift_left(labv - base, _POSBITS) | (iota + j * 16)
        plsc.store_compressed(hits_v.at[pl.ds(cnt, 16)], packed, mask=m)
        pc = plsc.all_reduce_population_count(m)
        return cnt + pc[0]

    cnt = lax.fori_loop(0, _BATCH // 16, scan_body, jnp.int32(0))
    cntv = jnp.broadcast_to(cnt, (16,))

    zeros = jnp.zeros((16,), jnp.float32)

    def process_chunk(g, buf, accs):
        c0l = g * _CHUNK - base

        def rescan(j, carry):
            ccnt = carry
            pv = hits_v[pl.ds(j * 16, 16)]
            labl = lax.shift_right_logical(pv, _POSBITS)
            m = ((iota + j * 16) < cntv) & (labl >= c0l) & (labl < c0l + _CHUNK)
            plsc.store_compressed(clist_v.at[pl.ds(ccnt, 16)], pv, mask=m)
            pc = plsc.all_reduce_population_count(m)
            return ccnt + pc[0]

        ccnt = lax.fori_loop(0, (cnt + 15) // 16, rescan, jnp.int32(0))

        def group(h, accs_):
            a0, a1, a2, a3 = accs_
            for k in range(8):
                e = jnp.minimum(h * 8 + k, ccnt - 1)
                pk = clist_v[pl.ds(e, 16)][0]
                pos = pk & (2 ** _POSBITS - 1)
                row0 = pl.multiple_of((pos >> 3) * 8, 8)
                pltpu.async_copy(feat_hbm.at[pl.ds(row0, 8)],
                                 ring_v.at[k], sem_feat)
            pltpu.make_async_copy(feat_hbm.at[pl.ds(0, 8 * 8)],
                                  _ring_flat(ring_v), sem_feat).wait()
            for k in range(8):
                e = h * 8 + k
                ec = jnp.minimum(e, ccnt - 1)
                pk = clist_v[pl.ds(ec, 16)][0]
                pos = pk & (2 ** _POSBITS - 1)
                col = lax.shift_right_logical(pk, _POSBITS) - c0l
                colv = jnp.broadcast_to(col, (16,))
                valid = (e < ccnt)
                r = pos & 7
                ds = []
                for q in range(4):
                    fv = ring_v[k, r, pl.ds(q * 16, 16)]
                    cv = plsc.load_gather(buf, [iota + q * 16, colv])
                    d = fv - cv
                    ds.append(jnp.where(valid, d * d, 0.0))
                a0 += ds[0]
                a1 += ds[1]
                a2 += ds[2]
                a3 += ds[3]
            return (a0, a1, a2, a3)

        return lax.fori_loop(0, (ccnt + 7) // 8, group, accs)

    def process_tail(accs):
        c0l = _TAIL0 - base

        def rescan(j, carry):
            ccnt = carry
            pv = hits_v[pl.ds(j * 16, 16)]
            labl = lax.shift_right_logical(pv, _POSBITS)
            m = ((iota + j * 16) < cntv) & (labl >= c0l)
            plsc.store_compressed(clist_v.at[pl.ds(ccnt, 16)], pv, mask=m)
            pc = plsc.all_reduce_population_count(m)
            return ccnt + pc[0]

        ccnt = lax.fori_loop(0, (cnt + 15) // 16, rescan, jnp.int32(0))

        def group(h, accs_):
            a0, a1, a2, a3 = accs_
            for k in range(8):
                e = jnp.minimum(h * 8 + k, ccnt - 1)
                pk = clist_v[pl.ds(e, 16)][0]
                pos = pk & (2 ** _POSBITS - 1)
                row0 = pl.multiple_of((pos >> 3) * 8, 8)
                pltpu.async_copy(feat_hbm.at[pl.ds(row0, 8)],
                                 ring_v.at[k], sem_feat)
            pltpu.make_async_copy(feat_hbm.at[pl.ds(0, 8 * 8)],
                                  _ring_flat(ring_v), sem_feat).wait()
            for k in range(8):
                e = h * 8 + k
                ec = jnp.minimum(e, ccnt - 1)
                pk = clist_v[pl.ds(ec, 16)][0]
                pos = pk & (2 ** _POSBITS - 1)
                col = lax.shift_right_logical(pk, _POSBITS) - c0l
                colv = jnp.broadcast_to(col, (16,))
                valid = (e < ccnt)
                r = pos & 7
                for q in range(4):
                    fv = ring_v[k, r, pl.ds(q * 16, 16)]
                    cv = plsc.load_gather(tail_v, [iota + q * 16, colv])
                    d = fv - cv
                    dd = jnp.where(valid, d * d, 0.0)
                    if q == 0:
                        a0 += dd
                    elif q == 1:
                        a1 += dd
                    elif q == 2:
                        a2 += dd
                    else:
                        a3 += dd
            return (a0, a1, a2, a3)

        return lax.fori_loop(0, (ccnt + 7) // 8, group, accs)

    npairs = (hi - lo + 1) // 2

    def pair_body(p, accs):
        g0 = lo + 2 * p
        g1 = g0 + 1
        pltpu.make_async_copy(
            centersT_hbm.at[:, pl.ds(0, _CHUNK)], buf0, sem0).wait()

        @pl.when(g1 < hi)
        def _():
            pltpu.async_copy(
                centersT_hbm.at[:, pl.ds(pl.multiple_of(g1 * _CHUNK, 128),
                                         _CHUNK)],
                buf1, sem1)

        accs = process_chunk(g0, buf0, accs)

        def odd(accs_):
            pltpu.make_async_copy(
                centersT_hbm.at[:, pl.ds(0, _CHUNK)], buf1, sem1).wait()

            @pl.when(g1 + 1 < hi)
            def _():
                pltpu.async_copy(
                    centersT_hbm.at[:, pl.ds(
                        pl.multiple_of((g1 + 1) * _CHUNK, 128), _CHUNK)],
                    buf0, sem0)

            return process_chunk(g1, buf1, accs_)

        return lax.cond(g1 < hi, odd, lambda a: a, accs)

    accs = lax.fori_loop(0, npairs, pair_body,
                         (zeros, zeros, zeros, zeros))

    accs = lax.cond(w == _NW - 1, process_tail, lambda a: a, accs)

    a0, a1, a2, a3 = accs
    part_v[...] = ((a0 + a1) + (a2 + a3)) * _SCALE
    pltpu.sync_copy(part_v, part_hbm.at[w])


def _ring_flat(ring_v):
    return ring_v


def _finish_body(part_ref, out_ref):
    out_ref[0, 0] = jnp.sum(part_ref[...])


def kernel(label, feat, centers):
    centersT = centers.T
    tail = lax.slice(centersT, (0, _TAIL0), (_FEAT, _NCLASS))

    sc = pl.kernel(
        _sc_body,
        out_type=jax.ShapeDtypeStruct((_NW, 16), jnp.float32),
        mesh=plsc.VectorSubcoreMesh(core_axis_name="c", subcore_axis_name="s"),
        compiler_params=pltpu.CompilerParams(needs_layout_passes=False),
        scratch_types=[
            pltpu.VMEM((_BATCH,), jnp.int32),
            pltpu.VMEM((_BATCH + 16,), jnp.int32),
            pltpu.VMEM((_BATCH + 16,), jnp.int32),
            pltpu.VMEM((_FEAT, _CHUNK), jnp.float32),
            pltpu.VMEM((_FEAT, _CHUNK), jnp.float32),
            pltpu.VMEM((8, 8, _FEAT), jnp.float32),
            pltpu.VMEM((_FEAT, _FEAT), jnp.float32),
            pltpu.VMEM((16,), jnp.float32),
            pltpu.SemaphoreType.DMA,
            pltpu.SemaphoreType.DMA,
            pltpu.SemaphoreType.DMA,
            pltpu.SemaphoreType.DMA,
        ],
    )
    partials = sc(label, feat, centersT, tail)

    loss11 = pl.pallas_call(
        _finish_body,
        out_shape=jax.ShapeDtypeStruct((1, 1), jnp.float32),
        out_specs=pl.BlockSpec(memory_space=pltpu.SMEM),
    )(partials)
    return loss11[0, 0]


def _run_unused():
    return kernel, (
        jax.ShapeDtypeStruct((_BATCH,), jnp.int32),
        jax.ShapeDtypeStruct((_BATCH, _FEAT), jnp.float32),
        jax.ShapeDtypeStruct((_NCLASS, _FEAT), jnp.float32),
    )


# speedup vs baseline: 3.3094x; 1.3464x over previous
import jax
import jax.numpy as jnp
from jax import lax
from jax.experimental import pallas as pl
from jax.experimental.pallas import tpu as pltpu
from jax.experimental.pallas import tpu_sc as plsc

_BATCH = 16384
_FEAT = 64
_NCLASS = 1000000
_NC = 2
_NS = 16
_NW = _NC * _NS
_CHUNK = 384
_NCHUNKS = 999936 // _CHUNK
_TAIL0 = 999936
_SCALE = 0.5 / _BATCH
_POSBITS = 14
_NSTAGE = 16


def _sc_body(label_hbm, centersT_hbm, tail_hbm, out_hbm,
             labels_v, hits_v, clist_v, buf0, buf1, stage_v, tail_v,
             sem_misc, sem_out, sem0, sem1):
    w = lax.axis_index("s") * _NC + lax.axis_index("c")
    lo = w * _NCHUNKS // _NW
    hi = (w + 1) * _NCHUNKS // _NW
    base = lo * _CHUNK
    limit = jnp.where(w == _NW - 1, _NCLASS, hi * _CHUNK)

    cp_lab = pltpu.async_copy(label_hbm, labels_v, sem_misc)
    cp_tail = pltpu.async_copy(tail_hbm, tail_v, sem_misc)
    pltpu.async_copy(
        centersT_hbm.at[:, pl.ds(pl.multiple_of(lo * _CHUNK, 128), _CHUNK)],
        buf0, sem0)
    cp_lab.wait()
    cp_tail.wait()

    iota = lax.iota(jnp.int32, 16)

    def scan_body(j, cnt):
        labv = labels_v[pl.ds(j * 16, 16)]
        m = (labv >= base) & (labv < limit)
        packed = lax.shift_left(labv - base, _POSBITS) | (iota + j * 16)
        plsc.store_compressed(hits_v.at[pl.ds(cnt, 16)], packed, mask=m)
        pc = plsc.all_reduce_population_count(m)
        return cnt + pc[0]

    cnt = lax.fori_loop(0, _BATCH // 16, scan_body, jnp.int32(0))
    cntv = jnp.broadcast_to(cnt, (16,))

    def emit_hits(buf, c0l, ccnt, nout0):
        def hit(e, nout):
            pk = clist_v[pl.ds(e, 16)][0]
            pos = pk & (2 ** _POSBITS - 1)
            col = lax.shift_right_logical(pk, _POSBITS) - c0l
            colv = jnp.broadcast_to(col, (16,))
            slot = nout % _NSTAGE
            soff = slot * _FEAT
            for q in range(4):
                cv = plsc.load_gather(buf, [iota + q * 16, colv])
                stage_v[pl.ds(soff + q * 16, 16)] = cv

            @pl.when(nout >= _NSTAGE)
            def _():
                pltpu.make_async_copy(
                    out_hbm.at[pl.ds(0, _FEAT)],
                    stage_v.at[pl.ds(0, _FEAT)], sem_out).wait()

            pltpu.async_copy(
                stage_v.at[pl.ds(soff, _FEAT)],
                out_hbm.at[pl.ds(pos * _FEAT, _FEAT)], sem_out)
            return nout + 1

        return lax.fori_loop(0, ccnt, hit, nout0)

    def rescan_range(lo_l, hi_l):
        def rescan(j, ccnt):
            pv = hits_v[pl.ds(j * 16, 16)]
            labl = lax.shift_right_logical(pv, _POSBITS)
            m = ((iota + j * 16) < cntv) & (labl >= lo_l) & (labl < hi_l)
            plsc.store_compressed(clist_v.at[pl.ds(ccnt, 16)], pv, mask=m)
            pc = plsc.all_reduce_population_count(m)
            return ccnt + pc[0]

        return lax.fori_loop(0, (cnt + 15) // 16, rescan, jnp.int32(0))

    def process_chunk(g, buf, nout):
        c0l = g * _CHUNK - base
        ccnt = rescan_range(c0l, c0l + _CHUNK)
        return emit_hits(buf, c0l, ccnt, nout)

    npairs = (hi - lo + 1) // 2

    def pair_body(p, nout):
        g0 = lo + 2 * p
        g1 = g0 + 1
        pltpu.make_async_copy(
            centersT_hbm.at[:, pl.ds(0, _CHUNK)], buf0, sem0).wait()

        @pl.when(g1 < hi)
        def _():
            pltpu.async_copy(
                centersT_hbm.at[:, pl.ds(pl.multiple_of(g1 * _CHUNK, 128),
                                         _CHUNK)],
                buf1, sem1)

        nout = process_chunk(g0, buf0, nout)

        def odd(nout_):
            pltpu.make_async_copy(
                centersT_hbm.at[:, pl.ds(0, _CHUNK)], buf1, sem1).wait()

            @pl.when(g1 + 1 < hi)
            def _():
                pltpu.async_copy(
                    centersT_hbm.at[:, pl.ds(
                        pl.multiple_of((g1 + 1) * _CHUNK, 128), _CHUNK)],
                    buf0, sem0)

            return process_chunk(g1, buf1, nout_)

        return lax.cond(g1 < hi, odd, lambda n: n, nout)

    nout = lax.fori_loop(0, npairs, pair_body, jnp.int32(0))

    def tail_hits(nout_):
        c0l = _TAIL0 - base
        ccnt = rescan_range(c0l, c0l + 2 ** (31 - _POSBITS))
        return emit_hits(tail_v, c0l, ccnt, nout_)

    nout = lax.cond(w == _NW - 1, tail_hits, lambda n: n, nout)

    def drain(_, __):
        pltpu.make_async_copy(
            out_hbm.at[pl.ds(0, _FEAT)],
            stage_v.at[pl.ds(0, _FEAT)], sem_out).wait()
        return __

    lax.fori_loop(0, jnp.minimum(nout, _NSTAGE), drain, jnp.int32(0))


def _finish_body(feat_ref, g_ref, out_ref):
    d = feat_ref[...] - g_ref[...]
    out_ref[0, 0] = jnp.sum(d * d) * _SCALE


@jax.jit
def kernel(label, feat, centers):
    centersT = centers.T
    tail = lax.slice(centersT, (0, _TAIL0), (_FEAT, _NCLASS))

    sc = pl.kernel(
        _sc_body,
        out_type=jax.ShapeDtypeStruct((_BATCH * _FEAT,), jnp.float32),
        mesh=plsc.VectorSubcoreMesh(core_axis_name="c", subcore_axis_name="s"),
        compiler_params=pltpu.CompilerParams(needs_layout_passes=False),
        scratch_types=[
            pltpu.VMEM((_BATCH,), jnp.int32),
            pltpu.VMEM((_BATCH + 16,), jnp.int32),
            pltpu.VMEM((_BATCH + 16,), jnp.int32),
            pltpu.VMEM((_FEAT, _CHUNK), jnp.float32),
            pltpu.VMEM((_FEAT, _CHUNK), jnp.float32),
            pltpu.VMEM((_NSTAGE * _FEAT,), jnp.float32),
            pltpu.VMEM((_FEAT, _FEAT), jnp.float32),
            pltpu.SemaphoreType.DMA,
            pltpu.SemaphoreType.DMA,
            pltpu.SemaphoreType.DMA,
            pltpu.SemaphoreType.DMA,
        ],
    )
    gathered = sc(label, centersT, tail)

    loss11 = pl.pallas_call(
        _finish_body,
        out_shape=jax.ShapeDtypeStruct((1, 1), jnp.float32),
        out_specs=pl.BlockSpec(memory_space=pltpu.SMEM),
    )(feat, gathered.reshape(_BATCH, _FEAT))
    return loss11[0, 0]


# speedup vs baseline: 3.5913x; 1.0852x over previous
import jax
import jax.numpy as jnp
from jax import lax
from jax.experimental import pallas as pl
from jax.experimental.pallas import tpu as pltpu
from jax.experimental.pallas import tpu_sc as plsc

_BATCH = 16384
_FEAT = 64
_NCLASS = 1000000
_NC = 2
_NS = 16
_NW = _NC * _NS
_CHUNK = 512
_NCHUNKS = 999936 // _CHUNK
_TAIL0 = 999936
_SCALE = 0.5 / _BATCH
_POSBITS = 14
_NSTAGE = 16


def _sc_body(label_hbm, centersT_hbm, tail_hbm, out_hbm,
             labels_v, hits_v, clist_v, buf0, buf1, stage_v, tail_v,
             sem_misc, sem_out, sem0, sem1):
    w = lax.axis_index("s") * _NC + lax.axis_index("c")
    lo = w * _NCHUNKS // _NW
    hi = (w + 1) * _NCHUNKS // _NW
    base = lo * _CHUNK
    limit = jnp.where(w == _NW - 1, _NCLASS, hi * _CHUNK)

    cp_lab = pltpu.async_copy(label_hbm, labels_v, sem_misc)
    cp_tail = pltpu.async_copy(tail_hbm, tail_v, sem_misc)
    pltpu.async_copy(
        centersT_hbm.at[:, pl.ds(pl.multiple_of(lo * _CHUNK, 128), _CHUNK)],
        buf0, sem0)
    cp_lab.wait()
    cp_tail.wait()

    iota = lax.iota(jnp.int32, 16)

    def scan_body(j, cnt):
        labv = labels_v[pl.ds(j * 16, 16)]
        m = (labv >= base) & (labv < limit)
        packed = lax.shift_left(labv - base, _POSBITS) | (iota + j * 16)
        plsc.store_compressed(hits_v.at[pl.ds(cnt, 16)], packed, mask=m)
        pc = plsc.all_reduce_population_count(m)
        return cnt + pc[0]

    cnt = lax.fori_loop(0, _BATCH // 16, scan_body, jnp.int32(0))
    cntv = jnp.broadcast_to(cnt, (16,))

    def emit_hits(buf, c0l, ccnt, nout0):
        def hit(e, nout):
            pk = clist_v[pl.ds(e, 16)][0]
            pos = pk & (2 ** _POSBITS - 1)
            col = lax.shift_right_logical(pk, _POSBITS) - c0l
            colv = jnp.broadcast_to(col, (16,))
            slot = nout % _NSTAGE
            soff = slot * _FEAT
            for q in range(4):
                cv = plsc.load_gather(buf, [iota + q * 16, colv])
                stage_v[pl.ds(soff + q * 16, 16)] = cv

            @pl.when(nout >= _NSTAGE)
            def _():
                pltpu.make_async_copy(
                    out_hbm.at[pl.ds(0, _FEAT)],
                    stage_v.at[pl.ds(0, _FEAT)], sem_out).wait()

            pltpu.async_copy(
                stage_v.at[pl.ds(soff, _FEAT)],
                out_hbm.at[pl.ds(pos * _FEAT, _FEAT)], sem_out)
            return nout + 1

        return lax.fori_loop(0, ccnt, hit, nout0)

    def rescan_range(lo_l, hi_l):
        def rescan(j, ccnt):
            pv = hits_v[pl.ds(j * 16, 16)]
            labl = lax.shift_right_logical(pv, _POSBITS)
            m = ((iota + j * 16) < cntv) & (labl >= lo_l) & (labl < hi_l)
            plsc.store_compressed(clist_v.at[pl.ds(ccnt, 16)], pv, mask=m)
            pc = plsc.all_reduce_population_count(m)
            return ccnt + pc[0]

        return lax.fori_loop(0, (cnt + 15) // 16, rescan, jnp.int32(0))

    def process_chunk(g, buf, nout):
        c0l = g * _CHUNK - base
        ccnt = rescan_range(c0l, c0l + _CHUNK)
        return emit_hits(buf, c0l, ccnt, nout)

    npairs = (hi - lo + 1) // 2

    def pair_body(p, nout):
        g0 = lo + 2 * p
        g1 = g0 + 1
        pltpu.make_async_copy(
            centersT_hbm.at[:, pl.ds(0, _CHUNK)], buf0, sem0).wait()

        @pl.when(g1 < hi)
        def _():
            pltpu.async_copy(
                centersT_hbm.at[:, pl.ds(pl.multiple_of(g1 * _CHUNK, 128),
                                         _CHUNK)],
                buf1, sem1)

        nout = process_chunk(g0, buf0, nout)

        def odd(nout_):
            pltpu.make_async_copy(
                centersT_hbm.at[:, pl.ds(0, _CHUNK)], buf1, sem1).wait()

            @pl.when(g1 + 1 < hi)
            def _():
                pltpu.async_copy(
                    centersT_hbm.at[:, pl.ds(
                        pl.multiple_of((g1 + 1) * _CHUNK, 128), _CHUNK)],
                    buf0, sem0)

            return process_chunk(g1, buf1, nout_)

        return lax.cond(g1 < hi, odd, lambda n: n, nout)

    nout = lax.fori_loop(0, npairs, pair_body, jnp.int32(0))

    def tail_hits(nout_):
        c0l = _TAIL0 - base
        ccnt = rescan_range(c0l, c0l + 2 ** (31 - _POSBITS))
        return emit_hits(tail_v, c0l, ccnt, nout_)

    nout = lax.cond(w == _NW - 1, tail_hits, lambda n: n, nout)

    def drain(_, __):
        pltpu.make_async_copy(
            out_hbm.at[pl.ds(0, _FEAT)],
            stage_v.at[pl.ds(0, _FEAT)], sem_out).wait()
        return __

    lax.fori_loop(0, jnp.minimum(nout, _NSTAGE), drain, jnp.int32(0))


def _finish_body(feat_ref, g_ref, out_ref):
    d = feat_ref[...] - g_ref[...]
    out_ref[0, 0] = jnp.sum(d * d) * _SCALE


@jax.jit
def kernel(label, feat, centers):
    centersT = centers.T
    tail = lax.slice(centersT, (0, _TAIL0), (_FEAT, _NCLASS))

    sc = pl.kernel(
        _sc_body,
        out_type=jax.ShapeDtypeStruct((_BATCH * _FEAT,), jnp.float32),
        mesh=plsc.VectorSubcoreMesh(core_axis_name="c", subcore_axis_name="s"),
        compiler_params=pltpu.CompilerParams(needs_layout_passes=False),
        scratch_types=[
            pltpu.VMEM((_BATCH,), jnp.int32),
            pltpu.VMEM((_BATCH + 16,), jnp.int32),
            pltpu.VMEM((_BATCH + 16,), jnp.int32),
            pltpu.VMEM((_FEAT, _CHUNK), jnp.float32),
            pltpu.VMEM((_FEAT, _CHUNK), jnp.float32),
            pltpu.VMEM((_NSTAGE * _FEAT,), jnp.float32),
            pltpu.VMEM((_FEAT, _FEAT), jnp.float32),
            pltpu.SemaphoreType.DMA,
            pltpu.SemaphoreType.DMA,
            pltpu.SemaphoreType.DMA,
            pltpu.SemaphoreType.DMA,
        ],
    )
    gathered = sc(label, centersT, tail)

    loss11 = pl.pallas_call(
        _finish_body,
        out_shape=jax.ShapeDtypeStruct((1, 1), jnp.float32),
        out_specs=pl.BlockSpec(memory_space=pltpu.SMEM),
    )(feat, gathered.reshape(_BATCH, _FEAT))
    return loss11[0, 0]
